# baseline (speedup 1.0000x reference)
import functools

import jax
import jax.numpy as jnp
from jax import lax
from jax.experimental import pallas as pl
from jax.experimental.pallas import tpu as pltpu

_NEG_BIG = -1e30


def _fused_attn_kernel(
    x_ref,
    wq_ref,
    bq_ref,
    wkv_ref,
    bkv_ref,
    mask_ref,
    wo_ref,
    bo_ref,
    out_ref,
    k_s,
    v_s,
    *,
    num_heads: int,
    block_k: int,
    proj_tile: int,
):
    tq = out_ref.shape[1]
    d_model = out_ref.shape[2]
    hd = d_model // num_heads
    qi = pl.program_id(1)
    q_start = qi * tq
    seq_len = x_ref.shape[1]

    @pl.when(qi == 0)
    def _project_kv():
        for r in range(seq_len // proj_tile):
            rows = slice(r * proj_tile, (r + 1) * proj_tile)
            acc = jnp.dot(x_ref[0, rows, :].astype(jnp.bfloat16),
                          wkv_ref[...], preferred_element_type=jnp.float32)
            acc = acc + bkv_ref[...]
            k_s[rows, :] = acc[:, :d_model].astype(jnp.bfloat16)
            v_s[rows, :] = acc[:, d_model:].astype(jnp.bfloat16)

    q_acc = jnp.dot(x_ref[0, pl.ds(q_start, tq), :].astype(jnp.bfloat16),
                    wq_ref[...], preferred_element_type=jnp.float32)
    q = (q_acc + bq_ref[...]).astype(jnp.bfloat16)

    def make_state():
        m = [jnp.full((tq, 1), -jnp.inf, jnp.float32) for _ in range(num_heads)]
        l = [jnp.zeros((tq, 1), jnp.float32) for _ in range(num_heads)]
        a = [jnp.zeros((tq, hd), jnp.float32) for _ in range(num_heads)]
        return tuple(m), tuple(l), tuple(a)

    def chunk_update(state, k_chunk, v_chunk, bias):
        m_t, l_t, a_t = state
        m_n, l_n, a_n = [], [], []
        for h in range(num_heads):
            sl = slice(h * hd, (h + 1) * hd)
            s = lax.dot_general(q[:, sl], k_chunk[:, sl],
                                (((1,), (1,)), ((), ())),
                                preferred_element_type=jnp.float32)
            s = s + bias
            m_new = jnp.maximum(m_t[h], jnp.max(s, axis=-1, keepdims=True))
            alpha = jnp.exp2(m_t[h] - m_new)
            p = jnp.exp2(s - m_new)
            l_n.append(alpha * l_t[h] + jnp.sum(p, axis=-1, keepdims=True))
            a_n.append(alpha * a_t[h] + jnp.dot(
                p.astype(jnp.bfloat16), v_chunk[:, sl],
                preferred_element_type=jnp.float32))
            m_n.append(m_new)
        return tuple(m_n), tuple(l_n), tuple(a_n)

    last = (q_start + tq - 1) // block_k

    def interior_body(i, state):
        off = i * block_k
        k_chunk = k_s[pl.ds(off, block_k), :]
        v_chunk = v_s[pl.ds(off, block_k), :]
        bias = mask_ref[0, pl.ds(i, 1), :]
        return chunk_update(state, k_chunk, v_chunk, bias)

    state = lax.fori_loop(0, last, interior_body, make_state())

    off = last * block_k
    k_chunk = k_s[pl.ds(off, block_k), :]
    v_chunk = v_s[pl.ds(off, block_k), :]
    row = lax.broadcasted_iota(jnp.int32, (tq, block_k), 0) + q_start
    col = lax.broadcasted_iota(jnp.int32, (tq, block_k), 1) + off
    bias = jnp.where(row >= col, mask_ref[0, pl.ds(last, 1), :],
                     jnp.float32(_NEG_BIG))
    m_t, l_t, a_t = chunk_update(state, k_chunk, v_chunk, bias)

    o = jnp.concatenate(
        [a_t[h] * (1.0 / l_t[h]) for h in range(num_heads)], axis=1)
    acc = jnp.dot(o.astype(jnp.bfloat16), wo_ref[...],
                  preferred_element_type=jnp.float32)
    out_ref[0] = (acc + bo_ref[...]).astype(out_ref.dtype)


def kernel(hidden_states, attention_mask, wq, bq, wk, bk, wv, bv, wo, bo):
    B, T, D = hidden_states.shape
    num_heads = 16
    head_dim = D // num_heads
    scaling = float(head_dim) ** -0.5

    log2e = 1.4426950408889634
    w_q = (wq.T * (scaling * log2e)).astype(jnp.bfloat16)
    b_q = (bq * (scaling * log2e)).reshape(1, D).astype(jnp.float32)
    w_kv = jnp.concatenate([wk.T, wv.T], axis=1).astype(jnp.bfloat16)
    b_kv = jnp.concatenate([bk, bv]).reshape(1, 2 * D).astype(jnp.float32)
    w_o = wo.T.astype(jnp.bfloat16)
    b_o = bo.reshape(1, D).astype(jnp.float32)

    tq = min(256, T)
    tk = min(256, T)
    key_mask = (attention_mask[:, 0, 0, :] * 1.4426950408889634).reshape(
        B, T // tk, tk)

    out = pl.pallas_call(
        functools.partial(_fused_attn_kernel, num_heads=num_heads,
                          block_k=tk, proj_tile=min(512, T)),
        out_shape=jax.ShapeDtypeStruct((B, T, D), hidden_states.dtype),
        grid_spec=pltpu.PrefetchScalarGridSpec(
            num_scalar_prefetch=0,
            grid=(B, T // tq),
            in_specs=[
                pl.BlockSpec((1, T, D), lambda b, qi: (b, 0, 0)),
                pl.BlockSpec((D, D), lambda b, qi: (0, 0)),
                pl.BlockSpec((1, D), lambda b, qi: (0, 0)),
                pl.BlockSpec((D, 2 * D), lambda b, qi: (0, 0)),
                pl.BlockSpec((1, 2 * D), lambda b, qi: (0, 0)),
                pl.BlockSpec((1, T // tk, tk), lambda b, qi: (b, 0, 0)),
                pl.BlockSpec((D, D), lambda b, qi: (0, 0)),
                pl.BlockSpec((1, D), lambda b, qi: (0, 0)),
            ],
            out_specs=pl.BlockSpec((1, tq, D), lambda b, qi: (b, qi, 0)),
            scratch_shapes=[
                pltpu.VMEM((T, D), jnp.bfloat16),
                pltpu.VMEM((T, D), jnp.bfloat16),
            ],
        ),
        compiler_params=pltpu.CompilerParams(
            dimension_semantics=("parallel", "arbitrary")),
    )(hidden_states, w_q, b_q, w_kv, b_kv, key_mask, w_o, b_o)

    return out


# speedup vs baseline: 1.2429x; 1.2429x over previous
import functools

import jax
import jax.numpy as jnp
from jax import lax
from jax.experimental import pallas as pl
from jax.experimental.pallas import tpu as pltpu

_NEG_BIG = -1e30


def _fused_attn_kernel(
    x_ref,
    wq_ref,
    bq_ref,
    wkv_ref,
    bkv_ref,
    mask_ref,
    wo_ref,
    bo_ref,
    out_ref,
    k_s,
    v_s,
    *,
    num_heads: int,
    block_k: int,
    proj_tile: int,
):
    tq = out_ref.shape[1]
    d_model = out_ref.shape[2]
    hd = d_model // num_heads
    qi = pl.program_id(1)
    q_start = qi * tq
    seq_len = x_ref.shape[1]

    @pl.when(qi == 0)
    def _project_kv():
        for r in range(seq_len // proj_tile):
            rows = slice(r * proj_tile, (r + 1) * proj_tile)
            acc = jnp.dot(x_ref[0, rows, :].astype(jnp.bfloat16),
                          wkv_ref[...], preferred_element_type=jnp.float32)
            acc = acc + bkv_ref[...]
            k_s[rows, :] = acc[:, :d_model].astype(jnp.bfloat16)
            v_s[rows, :] = acc[:, d_model:].astype(jnp.bfloat16)

    q_acc = jnp.dot(x_ref[0, pl.ds(q_start, tq), :].astype(jnp.bfloat16),
                    wq_ref[...], preferred_element_type=jnp.float32)
    q = (q_acc + bq_ref[...]).astype(jnp.bfloat16)

    def make_state():
        m = [jnp.full((tq, 1), -jnp.inf, jnp.float32) for _ in range(num_heads)]
        l = [jnp.zeros((tq, 1), jnp.float32) for _ in range(num_heads)]
        a = [jnp.zeros((tq, hd), jnp.float32) for _ in range(num_heads)]
        return tuple(m), tuple(l), tuple(a)

    def chunk_update(state, k_chunk, v_chunk, bias):
        m_t, l_t, a_t = state
        m_n, l_n, a_n = [], [], []
        for h in range(num_heads):
            sl = slice(h * hd, (h + 1) * hd)
            s = lax.dot_general(q[:, sl], k_chunk[:, sl],
                                (((1,), (1,)), ((), ())),
                                preferred_element_type=jnp.float32)
            s = s + bias
            m_new = jnp.maximum(m_t[h], jnp.max(s, axis=-1, keepdims=True))
            alpha = jnp.exp2(m_t[h] - m_new)
            p = jnp.exp2(s - m_new)
            l_n.append(alpha * l_t[h] + jnp.sum(p, axis=-1, keepdims=True))
            a_n.append(alpha * a_t[h] + jnp.dot(
                p.astype(jnp.bfloat16), v_chunk[:, sl],
                preferred_element_type=jnp.float32))
            m_n.append(m_new)
        return tuple(m_n), tuple(l_n), tuple(a_n)

    last = (q_start + tq - 1) // block_k

    def interior_body(i, state):
        off = i * block_k
        k_chunk = k_s[pl.ds(off, block_k), :]
        v_chunk = v_s[pl.ds(off, block_k), :]
        bias = mask_ref[0, pl.ds(i, 1), :]
        return chunk_update(state, k_chunk, v_chunk, bias)

    state = lax.fori_loop(0, last, interior_body, make_state())

    off = last * block_k
    k_chunk = k_s[pl.ds(off, block_k), :]
    v_chunk = v_s[pl.ds(off, block_k), :]
    row = lax.broadcasted_iota(jnp.int32, (tq, block_k), 0) + q_start
    col = lax.broadcasted_iota(jnp.int32, (tq, block_k), 1) + off
    bias = jnp.where(row >= col, mask_ref[0, pl.ds(last, 1), :],
                     jnp.float32(_NEG_BIG))
    m_t, l_t, a_t = chunk_update(state, k_chunk, v_chunk, bias)

    o = jnp.concatenate(
        [a_t[h] * (1.0 / l_t[h]) for h in range(num_heads)], axis=1)
    acc = jnp.dot(o.astype(jnp.bfloat16), wo_ref[...],
                  preferred_element_type=jnp.float32)
    out_ref[0] = (acc + bo_ref[...]).astype(out_ref.dtype)


def kernel(hidden_states, attention_mask, wq, bq, wk, bk, wv, bv, wo, bo):
    B, T, D = hidden_states.shape
    num_heads = 16
    head_dim = D // num_heads
    scaling = float(head_dim) ** -0.5

    log2e = 1.4426950408889634
    w_q = (wq.T * (scaling * log2e)).astype(jnp.bfloat16)
    b_q = (bq * (scaling * log2e)).reshape(1, D).astype(jnp.float32)
    w_kv = jnp.concatenate([wk.T, wv.T], axis=1).astype(jnp.bfloat16)
    b_kv = jnp.concatenate([bk, bv]).reshape(1, 2 * D).astype(jnp.float32)
    w_o = wo.T.astype(jnp.bfloat16)
    b_o = bo.reshape(1, D).astype(jnp.float32)

    tq = min(256, T)
    tk = min(512, T)
    key_mask = (attention_mask[:, 0, 0, :] * 1.4426950408889634).reshape(
        B, T // tk, tk)

    out = pl.pallas_call(
        functools.partial(_fused_attn_kernel, num_heads=num_heads,
                          block_k=tk, proj_tile=min(512, T)),
        out_shape=jax.ShapeDtypeStruct((B, T, D), hidden_states.dtype),
        grid_spec=pltpu.PrefetchScalarGridSpec(
            num_scalar_prefetch=0,
            grid=(B, T // tq),
            in_specs=[
                pl.BlockSpec((1, T, D), lambda b, qi: (b, 0, 0)),
                pl.BlockSpec((D, D), lambda b, qi: (0, 0)),
                pl.BlockSpec((1, D), lambda b, qi: (0, 0)),
                pl.BlockSpec((D, 2 * D), lambda b, qi: (0, 0)),
                pl.BlockSpec((1, 2 * D), lambda b, qi: (0, 0)),
                pl.BlockSpec((1, T // tk, tk), lambda b, qi: (b, 0, 0)),
                pl.BlockSpec((D, D), lambda b, qi: (0, 0)),
                pl.BlockSpec((1, D), lambda b, qi: (0, 0)),
            ],
            out_specs=pl.BlockSpec((1, tq, D), lambda b, qi: (b, qi, 0)),
            scratch_shapes=[
                pltpu.VMEM((T, D), jnp.bfloat16),
                pltpu.VMEM((T, D), jnp.bfloat16),
            ],
        ),
        compiler_params=pltpu.CompilerParams(
            dimension_semantics=("parallel", "arbitrary")),
    )(hidden_states, w_q, b_q, w_kv, b_kv, key_mask, w_o, b_o)

    return out
